# baseline (speedup 1.0000x reference)
import functools

import jax
import jax.numpy as jnp
from jax import lax
from jax.experimental import pallas as pl
from jax.experimental.pallas import tpu as pltpu
from jax.experimental.pallas import tpu_sc as plsc

BATCH = 16384
ACTIONS = 16

_info = plsc.get_sparse_core_info()
_NC, _NS, _L = _info.num_cores, _info.num_subcores, _info.num_lanes
_NW = _NC * _NS
_B_PER_W = BATCH // _NW
_CHUNK = 128
_NCHUNK = _B_PER_W // _CHUNK


def _sc_body(x_hbm, table_hbm, out_hbm, idx_v, rows_v, sem):
    wid = lax.axis_index("s") * _NC + lax.axis_index("c")
    base = wid * _B_PER_W

    pltpu.sync_copy(x_hbm.at[pl.ds(wid * _NCHUNK, _NCHUNK)], idx_v)

    copies = []
    for j in range(_NCHUNK):
        copies.append(
            pltpu.async_copy(
                table_hbm.at[idx_v.at[j]],
                rows_v.at[pl.ds(j * _CHUNK, _CHUNK)],
                sem,
            )
        )
    for c in copies:
        c.wait()

    lane = lax.iota(jnp.int32, _L)

    def softmax_block(blk, carry):
        row_ids = blk * _L + lane
        cols = [
            plsc.load_gather(rows_v, [row_ids, jnp.full((_L,), j, jnp.int32)])
            for j in range(ACTIONS)
        ]
        m = cols[0]
        for j in range(1, ACTIONS):
            m = jnp.maximum(m, cols[j])
        es = [jnp.exp(c - m) for c in cols]
        s = es[0]
        for j in range(1, ACTIONS):
            s = s + es[j]
        r = 1.0 / s
        for j in range(ACTIONS):
            plsc.store_scatter(
                rows_v, [row_ids, jnp.full((_L,), j, jnp.int32)], es[j] * r
            )
        return carry

    lax.fori_loop(0, _B_PER_W // _L, softmax_block, 0)

    pltpu.sync_copy(rows_v, out_hbm.at[pl.ds(base, _B_PER_W)])


@jax.jit
def _run(x, table):
    mesh = plsc.VectorSubcoreMesh(core_axis_name="c", subcore_axis_name="s")
    kern = functools.partial(
        pl.kernel,
        out_type=jax.ShapeDtypeStruct((BATCH, ACTIONS), jnp.float32),
        mesh=mesh,
        scratch_types=[
            pltpu.VMEM((_NCHUNK, _CHUNK), jnp.int32),
            pltpu.VMEM((_B_PER_W, ACTIONS), jnp.float32),
            pltpu.SemaphoreType.DMA,
        ],
        compiler_params=pltpu.CompilerParams(
            needs_layout_passes=False, use_tc_tiling_on_sc=False
        ),
    )(_sc_body)
    return kern(x, table)


def kernel(x, table):
    x = x.astype(jnp.int32).reshape(_NW * _NCHUNK, _CHUNK)
    return _run(x, table)
